# Initial kernel scaffold; baseline (speedup 1.0000x reference)
import jax
import jax.numpy as jnp
from jax.experimental import pallas as pl


def kernel(sample, entity_embedding, relation_embedding):
    raise NotImplementedError("write your pallas kernel here")


# speedup vs baseline: 1.4127x; 1.4127x over previous
import functools

import jax
import jax.numpy as jnp
from jax import lax
from jax.experimental import pallas as pl
from jax.experimental.pallas import tpu as pltpu
from jax.experimental.pallas import tpu_sc as plsc

GAMMA = 12.0


def _lane_shuffle(x, perm):
    dnums = lax.GatherDimensionNumbers(
        offset_dims=(), collapsed_slice_dims=(0,), start_index_map=(0,))
    return lax.gather(x, perm[:, None], dnums, (1,),
                      mode=lax.GatherScatterMode.PROMISE_IN_BOUNDS)


def _allsum(x, lanes):
    for k in (8, 4, 2, 1):
        x = x + _lane_shuffle(x, lanes ^ k)
    return x
B = 16384
D = 128
L = 16
NC, NS = 2, 16
NW = NC * NS
BPW = B // NW
C = 128
NCHUNK = BPW // C


def _build():
    mesh = plsc.VectorSubcoreMesh(core_axis_name="c", subcore_axis_name="s")

    @functools.partial(
        pl.kernel,
        mesh=mesh,
        out_type=jax.ShapeDtypeStruct((B,), jnp.float32),
        scratch_types=[
            pltpu.VMEM((C,), jnp.int32),
            pltpu.VMEM((C,), jnp.int32),
            pltpu.VMEM((C,), jnp.int32),
            pltpu.VMEM((C, D), jnp.float32),
            pltpu.VMEM((C, D), jnp.float32),
            pltpu.VMEM((C, D), jnp.float32),
            pltpu.VMEM((C,), jnp.float32),
            pltpu.SemaphoreType.DMA,
        ],
    )
    def k(hidx_hbm, ridx_hbm, tidx_hbm, ent_hbm, rel_hbm, out_hbm,
          hidx_v, ridx_v, tidx_v, hrow_v, rrow_v, trow_v, out_v, sem):
        wid = lax.axis_index("s") * NC + lax.axis_index("c")
        for c in range(NCHUNK):
            base = wid * BPW + c * C
            pltpu.sync_copy(hidx_hbm.at[pl.ds(base, C)], hidx_v)
            pltpu.sync_copy(ridx_hbm.at[pl.ds(base, C)], ridx_v)
            pltpu.sync_copy(tidx_hbm.at[pl.ds(base, C)], tidx_v)
            pltpu.async_copy(ent_hbm.at[hidx_v], hrow_v, sem).wait()
            pltpu.async_copy(rel_hbm.at[ridx_v], rrow_v, sem).wait()
            pltpu.async_copy(ent_hbm.at[tidx_v], trow_v, sem).wait()

            lanes = lax.iota(jnp.int32, L)

            def group(g, carry):
                vec = jnp.zeros((L,), jnp.float32)
                for r in range(L):
                    i = g * L + r
                    acc = jnp.zeros((L,), jnp.float32)
                    for p in range(D // L):
                        hv = hrow_v[i, pl.ds(p * L, L)]
                        rv = rrow_v[i, pl.ds(p * L, L)]
                        tv = trow_v[i, pl.ds(p * L, L)]
                        acc = acc + jnp.abs(hv + rv - tv)
                    vec = jnp.where(lanes == r, GAMMA - _allsum(acc, lanes), vec)
                out_v[pl.ds(g * L, L)] = vec
                return carry

            lax.fori_loop(0, C // L, group, 0)
            pltpu.sync_copy(out_v, out_hbm.at[pl.ds(base, C)])

    return k


_scored = _build()


def kernel(sample, entity_embedding, relation_embedding):
    s = sample.astype(jnp.int32)
    scores = _scored(s[:, 0], s[:, 1], s[:, 2],
                     entity_embedding, relation_embedding)
    return scores[:, None]


# speedup vs baseline: 1.8302x; 1.2956x over previous
import functools

import jax
import jax.numpy as jnp
from jax import lax
from jax.experimental import pallas as pl
from jax.experimental.pallas import tpu as pltpu
from jax.experimental.pallas import tpu_sc as plsc

GAMMA = 12.0
B = 16384
D = 128
L = 16
NC, NS = 2, 16
NW = NC * NS
BPW = B // NW
C = 128
NCHUNK = BPW // C
NBUF = 2


def _lane_shuffle(x, perm):
    dnums = lax.GatherDimensionNumbers(
        offset_dims=(), collapsed_slice_dims=(0,), start_index_map=(0,))
    return lax.gather(x, perm[:, None], dnums, (1,),
                      mode=lax.GatherScatterMode.PROMISE_IN_BOUNDS)


def _allsum(x, lanes):
    for k in (8, 4, 2, 1):
        x = x + _lane_shuffle(x, lanes ^ k)
    return x


def _build():
    mesh = plsc.VectorSubcoreMesh(core_axis_name="c", subcore_axis_name="s")

    @functools.partial(
        pl.kernel,
        mesh=mesh,
        out_type=jax.ShapeDtypeStruct((B,), jnp.float32),
        scratch_types=[
            pltpu.VMEM((BPW,), jnp.int32),
            pltpu.VMEM((BPW,), jnp.int32),
            pltpu.VMEM((BPW,), jnp.int32),
            pltpu.VMEM((NBUF, C, D), jnp.float32),
            pltpu.VMEM((NBUF, C, D), jnp.float32),
            pltpu.VMEM((NBUF, C, D), jnp.float32),
            pltpu.VMEM((BPW,), jnp.float32),
            pltpu.SemaphoreType.DMA,
            pltpu.SemaphoreType.DMA,
        ],
    )
    def k(hidx_hbm, ridx_hbm, tidx_hbm, ent_hbm, rel_hbm, out_hbm,
          hidx_v, ridx_v, tidx_v, hrow_v, rrow_v, trow_v, out_v, sem0, sem1):
        wid = lax.axis_index("s") * NC + lax.axis_index("c")
        base = wid * BPW
        pltpu.sync_copy(hidx_hbm.at[pl.ds(base, BPW)], hidx_v)
        pltpu.sync_copy(ridx_hbm.at[pl.ds(base, BPW)], ridx_v)
        pltpu.sync_copy(tidx_hbm.at[pl.ds(base, BPW)], tidx_v)
        sems = (sem0, sem1)

        def fire(c):
            buf = c % NBUF
            sl = pl.ds(c * C, C)
            sem = sems[buf]
            return (
                pltpu.async_copy(ent_hbm.at[hidx_v.at[sl]], hrow_v.at[buf], sem),
                pltpu.async_copy(rel_hbm.at[ridx_v.at[sl]], rrow_v.at[buf], sem),
                pltpu.async_copy(ent_hbm.at[tidx_v.at[sl]], trow_v.at[buf], sem),
            )

        pending = fire(0)
        lanes = lax.iota(jnp.int32, L)
        for c in range(NCHUNK):
            buf = c % NBUF
            nxt = fire(c + 1) if c + 1 < NCHUNK else None
            for cp in pending:
                cp.wait()
            pending = nxt

            @plsc.parallel_loop(0, C // L, unroll=1)
            def group(g):
                vec = jnp.zeros((L,), jnp.float32)
                for r in range(L):
                    i = g * L + r
                    acc = jnp.zeros((L,), jnp.float32)
                    for p in range(D // L):
                        hv = hrow_v[buf, i, pl.ds(p * L, L)]
                        rv = rrow_v[buf, i, pl.ds(p * L, L)]
                        tv = trow_v[buf, i, pl.ds(p * L, L)]
                        acc = acc + jnp.abs(hv + rv - tv)
                    vec = jnp.where(lanes == r, GAMMA - _allsum(acc, lanes), vec)
                out_v[pl.ds(c * C + g * L, L)] = vec

        pltpu.sync_copy(out_v, out_hbm.at[pl.ds(base, BPW)])

    return k


_scored = _build()


def kernel(sample, entity_embedding, relation_embedding):
    s = sample.astype(jnp.int32)
    scores = _scored(s[:, 0], s[:, 1], s[:, 2],
                     entity_embedding, relation_embedding)
    return scores[:, None]
